# baseline (speedup 1.0000x reference)
import functools

import jax
import jax.numpy as jnp
from jax import lax
from jax.experimental import pallas as pl
from jax.experimental.pallas import tpu as pltpu
from jax.experimental.pallas import tpu_sc as plsc

_V = 201
_VPAD = 208
_D = 64 * 64
_B = 16384
_K = 8


@functools.lru_cache(maxsize=None)
def _make_gather():
    info = plsc.get_sparse_core_info()
    nw = info.num_cores * info.num_subcores
    b_per_w = _B // nw
    nchunks = b_per_w // _K
    mesh = plsc.VectorSubcoreMesh(core_axis_name="c", subcore_axis_name="s")

    @functools.partial(
        pl.kernel,
        out_type=jax.ShapeDtypeStruct((_B * _D,), jnp.float32),
        mesh=mesh,
        scratch_types=[
            pltpu.VMEM_SHARED((_VPAD * _D,), jnp.float32),
            pltpu.VMEM_SHARED((_B,), jnp.int32),
            pltpu.SMEM((b_per_w,), jnp.int32),
            pltpu.VMEM((_K * _D,), jnp.float32),
            pltpu.VMEM((_K * _D,), jnp.float32),
            pltpu.SemaphoreType.DMA,
            pltpu.SemaphoreType.DMA,
            pltpu.SemaphoreType.DMA,
            pltpu.SemaphoreType.DMA,
        ],
    )
    def gather(idx_hbm, table_hbm, out_hbm, table_sh, idx_sh, idx_s,
               buf0, buf1, semr0, semr1, semw0, semw1):
        sid = lax.axis_index("s")
        wid = sid * info.num_cores + lax.axis_index("c")
        base = wid * b_per_w

        nstripes = _VPAD // 16
        fill_off = pl.multiple_of(sid * 16 * _D, 8)

        @pl.when(sid < nstripes)
        def _fill():
            pltpu.sync_copy(
                table_hbm.at[pl.ds(fill_off, 16 * _D)],
                table_sh.at[pl.ds(fill_off, 16 * _D)],
            )

        @pl.when(sid == 0)
        def _fill_idx():
            pltpu.sync_copy(idx_hbm, idx_sh)

        plsc.subcore_barrier()
        pltpu.sync_copy(idx_sh.at[pl.ds(base, b_per_w)], idx_s)

        bufs = (buf0, buf1)
        semrs = (semr0, semr1)
        semws = (semw0, semw1)

        def read_chunk(c, buf, semr):
            for j in range(_K):
                src_off = pl.multiple_of(idx_s[c * _K + j] * _D, 8)
                pltpu.async_copy(
                    table_sh.at[pl.ds(src_off, _D)],
                    buf.at[pl.ds(j * _D, _D)],
                    semr,
                )

        def drain_chunk(buf, semr):
            pltpu.make_async_copy(
                table_sh.at[pl.ds(0, _K * _D)], buf, semr
            ).wait()

        def write_chunk(c, buf, semw):
            dst = pl.multiple_of((base + c * _K) * _D, 8)
            pltpu.async_copy(buf, out_hbm.at[pl.ds(dst, _K * _D)], semw)

        def wait_write(c, buf, semw):
            dst = pl.multiple_of((base + c * _K) * _D, 8)
            pltpu.make_async_copy(
                buf, out_hbm.at[pl.ds(dst, _K * _D)], semw
            ).wait()

        def body(p, carry):
            for b in range(2):
                c = p * 2 + b
                pl.when(c >= 2)(
                    lambda: wait_write(c - 2, bufs[b], semws[b])
                )
                read_chunk(c, bufs[b], semrs[b])
                drain_chunk(bufs[b], semrs[b])
                write_chunk(c, bufs[b], semws[b])
            return carry

        lax.fori_loop(0, nchunks // 2, body, 0)
        wait_write(nchunks - 2, bufs[0], semws[0])
        wait_write(nchunks - 1, bufs[1], semws[1])

    return gather


def kernel(x, weights):
    table = weights.reshape(_V, _D)
    table = jnp.pad(table, ((0, _VPAD - _V), (0, 0)))
    out = _make_gather()(x, table.reshape(-1))
    return out.reshape(_B, _D)


# speedup vs baseline: 2.0985x; 2.0985x over previous
import functools

import jax
import jax.numpy as jnp
from jax import lax
from jax.experimental import pallas as pl
from jax.experimental.pallas import tpu as pltpu
from jax.experimental.pallas import tpu_sc as plsc

_V = 201
_VPAD = 208
_D = 64 * 64
_B = 16384
_K = 8
_PATTERN = (False, False, True, False, False, True, False, True,
            False, False, True, False, True, False, False, True)


@functools.lru_cache(maxsize=None)
def _make_gather():
    info = plsc.get_sparse_core_info()
    nw = info.num_cores * info.num_subcores
    b_per_w = _B // nw
    nchunks = b_per_w // _K
    period = len(_PATTERN)
    nper = nchunks // period
    mesh = plsc.VectorSubcoreMesh(core_axis_name="c", subcore_axis_name="s")

    @functools.partial(
        pl.kernel,
        out_type=jax.ShapeDtypeStruct((_B, _D), jnp.float32),
        mesh=mesh,
        scratch_types=[
            pltpu.VMEM_SHARED((_VPAD * _D,), jnp.float32),
            pltpu.VMEM_SHARED((_B,), jnp.int32),
            pltpu.SMEM((b_per_w,), jnp.int32),
            pltpu.VMEM((b_per_w,), jnp.int32),
            pltpu.VMEM((_K, _D), jnp.float32),
            pltpu.VMEM((_K, _D), jnp.float32),
            pltpu.SemaphoreType.DMA,
            pltpu.SemaphoreType.DMA,
            pltpu.SemaphoreType.DMA,
            pltpu.SemaphoreType.DMA,
            pltpu.SemaphoreType.DMA,
        ],
    )
    def gather(idx_hbm, tabf_hbm, tab2_hbm, out_hbm, table_sh, idx_sh, idx_s,
               idx_v, buf0, buf1, semr0, semr1, semw0, semw1, semd):
        sid = lax.axis_index("s")
        wid = sid * info.num_cores + lax.axis_index("c")
        base = wid * b_per_w

        nstripes = _VPAD // 16
        fill_off = pl.multiple_of(sid * 16 * _D, 8)

        @pl.when(sid < nstripes)
        def _fill():
            pltpu.sync_copy(
                tabf_hbm.at[pl.ds(fill_off, 16 * _D)],
                table_sh.at[pl.ds(fill_off, 16 * _D)],
            )

        @pl.when(sid == 0)
        def _fill_idx():
            pltpu.sync_copy(idx_hbm, idx_sh)

        pltpu.sync_copy(idx_hbm.at[pl.ds(base, b_per_w)], idx_v)
        plsc.subcore_barrier()
        pltpu.sync_copy(idx_sh.at[pl.ds(base, b_per_w)], idx_s)

        bufs = (buf0, buf1)
        semrs = (semr0, semr1)
        semws = (semw0, semw1)

        def issue_gather(c, b):
            off = pl.multiple_of(c * _K, 8)
            pltpu.async_copy(
                tab2_hbm.at[idx_v.at[pl.ds(off, _K)]], bufs[b], semrs[b]
            )

        def drain_gather(c, b):
            off = pl.multiple_of(c * _K, 8)
            pltpu.make_async_copy(
                tab2_hbm.at[idx_v.at[pl.ds(off, _K)]], bufs[b], semrs[b]
            ).wait()

        def issue_write(c, b):
            dst = pl.multiple_of(base + c * _K, 8)
            pltpu.async_copy(bufs[b], out_hbm.at[pl.ds(dst, _K)], semws[b])

        def wait_write(b):
            dst = pl.multiple_of(base, 8)
            pltpu.make_async_copy(
                bufs[b], out_hbm.at[pl.ds(dst, _K)], semws[b]
            ).wait()

        def issue_direct(c):
            for j in range(_K):
                src = pl.multiple_of(idx_s[c * _K + j] * _D, 8)
                pltpu.async_copy(
                    table_sh.at[pl.ds(src, _D)],
                    out_hbm.at[base + c * _K + j],
                    semd,
                )

        def drain_direct():
            for _ in range(_K):
                pltpu.make_async_copy(
                    table_sh.at[pl.ds(0, _D)], out_hbm.at[base], semd
                ).wait()

        def body(p, carry):
            sb = 0
            first_use = [True, True]
            first_d = True
            for q in range(period):
                c = p * period + q
                if _PATTERN[q]:
                    if first_d:
                        pl.when(p > 0)(drain_direct)
                        first_d = False
                    else:
                        drain_direct()
                    issue_direct(c)
                else:
                    b = sb % 2
                    sb += 1
                    if first_use[b]:
                        pl.when(p > 0)(lambda b=b: wait_write(b))
                        first_use[b] = False
                    else:
                        wait_write(b)
                    issue_gather(c, b)
                    drain_gather(c, b)
                    issue_write(c, b)
            return carry

        lax.fori_loop(0, nper, body, 0)
        drain_direct()
        wait_write(0)
        wait_write(1)

    return gather


def kernel(x, weights):
    table = weights.reshape(_V, _D)
    table = jnp.pad(table, ((0, _VPAD - _V), (0, 0)))
    out = _make_gather()(x, table.reshape(-1), table)
    return out
